# Initial kernel scaffold; baseline (speedup 1.0000x reference)
import jax
import jax.numpy as jnp
from jax.experimental import pallas as pl


def kernel(x, edge_index, batch, l0_w1, l0_b1, l0_g, l0_be, l0_w2, l0_b2, l1_w1, l1_b1, l1_g, l1_be, l1_w2, l1_b2, l2_w1, l2_b1, l2_g, l2_be, l2_w2, l2_b2, cls_w, cls_b):
    raise NotImplementedError("write your pallas kernel here")


# speedup vs baseline: 1.9573x; 1.9573x over previous
import functools

import jax
import jax.numpy as jnp
from jax import lax
from jax.experimental import pallas as pl
from jax.experimental.pallas import tpu as pltpu
from jax.experimental.pallas import tpu_sc as plsc

NN = 50000
EE = 800000
HH = 128
GG = 512

NC, NS = 2, 16
CH = 128
KG = 8
CPT = 400
E_PAD1 = NS * CPT * CH
CPT0 = 208
E_PAD0 = 2 * NS * CPT0 * CH
NPART = 8
PW = HH // NPART
RPT = NN // NS
NBLK = 50
BN = NN // NBLK

_f32 = jnp.float32


def _sc_mesh():
    return plsc.VectorSubcoreMesh(
        core_axis_name="c", subcore_axis_name="s", num_cores=NC, num_subcores=NS)


def _edge_pipeline(hqr, srow, drow, acc, bufs, sidx, didx, sems, n_chunks):
    ngroups = n_chunks // KG

    def grp(g, carry):
        pltpu.sync_copy(srow.at[pl.ds(g * KG, KG)], sidx.at[0])
        pltpu.sync_copy(drow.at[pl.ds(g * KG, KG)], didx.at[0])
        for b in range(KG):
            pltpu.sync_copy(hqr.at[sidx.at[0, b]], bufs.at[0, b])
            pltpu.sync_copy(bufs.at[0, b], acc.at[didx.at[0, b]], add=True)
        return carry

    lax.fori_loop(0, ngroups, grp, 0)


def _sc_agg_parts(h_all, src_h, dst_h):
    @functools.partial(
        pl.kernel,
        out_type=jax.ShapeDtypeStruct((NPART, NN, PW), _f32),
        mesh=_sc_mesh(),
        compiler_params=pltpu.CompilerParams(use_tc_tiling_on_sc=False),
        scratch_types=[
            pltpu.VMEM((2, KG, CH, PW), _f32),
            pltpu.VMEM((2, KG, CH), jnp.int32),
            pltpu.VMEM((2, KG, CH), jnp.int32),
            pltpu.VMEM_SHARED((NN + 8, PW), _f32),
        ] + [pltpu.SemaphoreType.DMA] * 8,
    )
    def k(hr, srcr, dstr, outr, bufs, sidx, didx, acc, *sems):
        c = lax.axis_index("c")
        s = lax.axis_index("s")
        srow = srcr.at[s]
        drow = dstr.at[s]
        r0 = s * RPT

        def one_pass(p, carry):
            q = 2 * p + c
            hqr = hr.at[q]
            oqr = outr.at[q]
            pltpu.sync_copy(hqr.at[pl.ds(r0, RPT)], acc.at[pl.ds(r0, RPT)])
            plsc.subcore_barrier()
            _edge_pipeline(hqr, srow, drow, acc, bufs, sidx, didx, sems, CPT)
            plsc.subcore_barrier()
            pltpu.sync_copy(acc.at[pl.ds(r0, RPT)], oqr.at[pl.ds(r0, RPT)])
            plsc.subcore_barrier()
            return carry

        lax.fori_loop(0, NPART // 2, one_pass, 0)

    return k(h_all, src_h, dst_h)


def _sc_agg_l0(x, z2, src_h, dst_h):
    @functools.partial(
        pl.kernel,
        out_type=[jax.ShapeDtypeStruct((NN, 16), _f32)] * 2,
        mesh=_sc_mesh(),
        compiler_params=pltpu.CompilerParams(use_tc_tiling_on_sc=False),
        scratch_types=[
            pltpu.VMEM((2, KG, CH, 16), _f32),
            pltpu.VMEM((2, KG, CH), jnp.int32),
            pltpu.VMEM((2, KG, CH), jnp.int32),
            pltpu.VMEM_SHARED((NN + 8, 16), _f32),
        ] + [pltpu.SemaphoreType.DMA] * 8,
    )
    def k(xr, zr, srcr, dstr, o0, o1, bufs, sidx, didx, acc, *sems):
        c = lax.axis_index("c")
        s = lax.axis_index("s")
        w = c * NS + s
        srow = srcr.at[w]
        drow = dstr.at[w]
        r0 = s * RPT
        for cc in range(2):
            init = (xr, zr)[cc]
            out = (o0, o1)[cc]

            @pl.when(c == cc)
            def _(init=init, out=out):
                pltpu.sync_copy(init.at[pl.ds(r0, RPT)], acc.at[pl.ds(r0, RPT)])
                plsc.subcore_barrier()
                _edge_pipeline(xr, srow, drow, acc, bufs, sidx, didx, sems, CPT0)
                plsc.subcore_barrier()
                pltpu.sync_copy(acc.at[pl.ds(r0, RPT)], out.at[pl.ds(r0, RPT)])

    return k(x, z2, src_h, dst_h)


def _full(shape):
    return pl.BlockSpec(shape, lambda i: tuple(0 for _ in shape))


def _sparts_specs(sparts):
    if isinstance(sparts, (list, tuple)):
        n = len(sparts)
        fin = sparts[0].shape[1]
        specs = [pl.BlockSpec((BN, fin), lambda i: (i, 0))] * n

        def read(refs):
            s = refs[0][...]
            for r in refs[1:]:
                s = s + r[...]
            return s

        return list(sparts), specs, n, read
    specs = [pl.BlockSpec((NPART, BN, PW), lambda i: (0, i, 0))]

    def read(refs):
        return jnp.concatenate([refs[0][qi] for qi in range(NPART)], axis=1)

    return [sparts], specs, 1, read


def _tc_stats(sparts, w1, b1):
    arrs, specs, n, read = _sparts_specs(sparts)

    def body(*refs):
        s = read(refs[:n])
        h1 = jnp.dot(s, refs[n][...], preferred_element_type=_f32) + refs[n + 1][...]
        ps = jnp.sum(h1, axis=0, keepdims=True)
        pq = jnp.sum(h1 * h1, axis=0, keepdims=True)
        stats = refs[n + 2]

        @pl.when(pl.program_id(0) == 0)
        def _():
            stats[...] = jnp.zeros_like(stats)

        stats[...] += jnp.concatenate([ps, pq, jnp.zeros((6, HH), _f32)], axis=0)

    return pl.pallas_call(
        body,
        grid=(NBLK,),
        in_specs=specs + [_full(w1.shape), _full((1, HH))],
        out_specs=_full((8, HH)),
        out_shape=jax.ShapeDtypeStruct((8, HH), _f32),
        compiler_params=pltpu.CompilerParams(dimension_semantics=("arbitrary",)),
    )(*arrs, w1, b1.reshape(1, HH))


def _bn_coeffs(statsr, gr, ber):
    st = statsr[...]
    mu = st[0:1, :] * (1.0 / NN)
    ex2 = st[1:2, :] * (1.0 / NN)
    var = ex2 - mu * mu
    sc = gr[...] * lax.rsqrt(var + 1e-5)
    sh = ber[...] - mu * sc
    return sc, sh


def _tc_mlp(sparts, stats, w1, b1, g, be, w2, b2):
    arrs, specs, n, read = _sparts_specs(sparts)

    def body(*refs):
        s = read(refs[:n])
        statsr, w1r, b1r, gr, ber, w2r, b2r, outr = refs[n:]
        sc, sh = _bn_coeffs(statsr, gr, ber)
        h1 = jnp.dot(s, w1r[...], preferred_element_type=_f32) + b1r[...]
        h = jnp.maximum(h1 * sc + sh, 0.0)
        h2 = jnp.maximum(jnp.dot(h, w2r[...], preferred_element_type=_f32) + b2r[...], 0.0)
        for qi in range(NPART):
            outr[qi] = h2[:, qi * PW:(qi + 1) * PW]

    return pl.pallas_call(
        body,
        grid=(NBLK,),
        in_specs=specs + [_full((8, HH)), _full(w1.shape), _full((1, HH)),
                          _full((1, HH)), _full((1, HH)), _full((HH, HH)),
                          _full((1, HH))],
        out_specs=pl.BlockSpec((NPART, BN, PW), lambda i: (0, i, 0)),
        out_shape=jax.ShapeDtypeStruct((NPART, NN, PW), _f32),
        compiler_params=pltpu.CompilerParams(dimension_semantics=("arbitrary",)),
    )(*arrs, stats, w1, b1.reshape(1, HH), g.reshape(1, HH), be.reshape(1, HH),
      w2, b2.reshape(1, HH))


def _tc_mlp_pool(sparts, stats, w1, b1, g, be, w2, b2, batch3, cls_w, cls_b):
    arrs, specs, n, read = _sparts_specs(sparts)

    def body(*refs):
        s = read(refs[:n])
        statsr, w1r, b1r, gr, ber, w2r, b2r, batchr, cwr, cbr, outv, pool = refs[n:]
        i = pl.program_id(0)
        sc, sh = _bn_coeffs(statsr, gr, ber)
        h1 = jnp.dot(s, w1r[...], preferred_element_type=_f32) + b1r[...]
        h = jnp.maximum(h1 * sc + sh, 0.0)
        h2 = jnp.maximum(jnp.dot(h, w2r[...], preferred_element_type=_f32) + b2r[...], 0.0)
        b = batchr[0, 0, :]
        onehot = (b[:, None] == lax.broadcasted_iota(jnp.int32, (BN, GG), 1)).astype(_f32)
        pblk = lax.dot_general(onehot, h2, (((0,), (0,)), ((), ())),
                               preferred_element_type=_f32)

        @pl.when(i == 0)
        def _():
            pool[...] = jnp.zeros_like(pool)

        pool[...] += pblk
        outv[...] = jnp.dot(pool[...], cwr[...], preferred_element_type=_f32) + cbr[...]

    return pl.pallas_call(
        body,
        grid=(NBLK,),
        in_specs=specs + [_full((8, HH)), _full((HH, HH)), _full((1, HH)),
                          _full((1, HH)), _full((1, HH)), _full((HH, HH)),
                          _full((1, HH)),
                          pl.BlockSpec((1, 1, BN), lambda i: (i, 0, 0)),
                          _full((HH, 1)), _full((1, 1))],
        out_specs=_full((GG, 1)),
        out_shape=jax.ShapeDtypeStruct((GG, 1), _f32),
        scratch_shapes=[pltpu.VMEM((GG, HH), _f32)],
        compiler_params=pltpu.CompilerParams(dimension_semantics=("arbitrary",)),
    )(*arrs, stats, w1, b1.reshape(1, HH), g.reshape(1, HH), be.reshape(1, HH),
      w2, b2.reshape(1, HH), batch3, cls_w, cls_b.reshape(1, 1))


def kernel(x, edge_index, batch, l0_w1, l0_b1, l0_g, l0_be, l0_w2, l0_b2,
           l1_w1, l1_b1, l1_g, l1_be, l1_w2, l1_b2,
           l2_w1, l2_b1, l2_g, l2_be, l2_w2, l2_b2, cls_w, cls_b):
    src = edge_index[0]
    dst = edge_index[1]
    i32 = jnp.int32
    src1 = jnp.concatenate([src, jnp.zeros((E_PAD1 - EE,), i32)]).reshape(NS, CPT, CH)
    dst1 = jnp.concatenate([dst, jnp.full((E_PAD1 - EE,), NN, i32)]).reshape(NS, CPT, CH)
    src0 = jnp.concatenate([src, jnp.zeros((E_PAD0 - EE,), i32)]).reshape(2 * NS, CPT0, CH)
    dst0 = jnp.concatenate([dst, jnp.full((E_PAD0 - EE,), NN, i32)]).reshape(2 * NS, CPT0, CH)
    x16 = jnp.pad(x, ((0, 0), (0, 14)))
    z16 = jnp.zeros((NN, 16), _f32)
    w1p = jnp.pad(l0_w1, ((0, 14), (0, 0)))
    batch3 = batch.reshape(NBLK, 1, BN)

    part0, part1 = _sc_agg_l0(x16, z16, src0, dst0)
    st0 = _tc_stats([part0, part1], w1p, l0_b1)
    h_all = _tc_mlp([part0, part1], st0, w1p, l0_b1, l0_g, l0_be, l0_w2, l0_b2)

    s_all = _sc_agg_parts(h_all, src1, dst1)
    st1 = _tc_stats(s_all, l1_w1, l1_b1)
    h_all = _tc_mlp(s_all, st1, l1_w1, l1_b1, l1_g, l1_be, l1_w2, l1_b2)

    s_all = _sc_agg_parts(h_all, src1, dst1)
    st2 = _tc_stats(s_all, l2_w1, l2_b1)
    outv = _tc_mlp_pool(s_all, st2, l2_w1, l2_b1, l2_g, l2_be, l2_w2, l2_b2,
                        batch3, cls_w, cls_b)
    return outv.reshape(-1)


# speedup vs baseline: 2.8881x; 1.4756x over previous
import functools

import jax
import jax.numpy as jnp
from jax import lax
from jax.experimental import pallas as pl
from jax.experimental.pallas import tpu as pltpu
from jax.experimental.pallas import tpu_sc as plsc

NN = 50000
EE = 800000
HH = 128
GG = 512

NC, NS = 2, 16
CH = 128
KG = 8
CPT = 400
E_PAD1 = NS * CPT * CH
CPT0 = 208
E_PAD0 = 2 * NS * CPT0 * CH
NPART = 8
PW = HH // NPART
RPT = NN // NS
NBLK = 50
BN = NN // NBLK

_f32 = jnp.float32


def _sc_mesh():
    return plsc.VectorSubcoreMesh(
        core_axis_name="c", subcore_axis_name="s", num_cores=NC, num_subcores=NS)


def _edge_pipeline(hqr, srow, drow, acc, bufs, sidx, didx, sems, n_chunks):
    gs0, gs1, ss0, ss1 = sems[:4]
    npairs = n_chunks // (2 * KG)

    def grp(gp, carry):
        ga = gp * 2
        pltpu.sync_copy(srow.at[pl.ds(ga * KG, KG)], sidx.at[0])
        pltpu.sync_copy(srow.at[pl.ds((ga + 1) * KG, KG)], sidx.at[1])
        pltpu.sync_copy(drow.at[pl.ds(ga * KG, KG)], didx.at[0])
        pltpu.sync_copy(drow.at[pl.ds((ga + 1) * KG, KG)], didx.at[1])
        dga = [pltpu.async_copy(hqr.at[sidx.at[0, b]], bufs.at[0, b], gs0)
               for b in range(KG)]
        dgb = [pltpu.async_copy(hqr.at[sidx.at[1, b]], bufs.at[1, b], gs1)
               for b in range(KG)]
        for d in dga:
            d.wait()
        dsa = [pltpu.async_copy(bufs.at[0, b], acc.at[didx.at[0, b]], ss0,
                                add=True) for b in range(KG)]
        for d in dgb:
            d.wait()
        dsb = [pltpu.async_copy(bufs.at[1, b], acc.at[didx.at[1, b]], ss1,
                                add=True) for b in range(KG)]
        for d in dsa:
            d.wait()
        for d in dsb:
            d.wait()
        return carry

    lax.fori_loop(0, npairs, grp, 0)


def _sc_agg_parts(h_all, src_h, dst_h):
    @functools.partial(
        pl.kernel,
        out_type=jax.ShapeDtypeStruct((NPART, NN, PW), _f32),
        mesh=_sc_mesh(),
        compiler_params=pltpu.CompilerParams(use_tc_tiling_on_sc=False),
        scratch_types=[
            pltpu.VMEM((2, KG, CH, PW), _f32),
            pltpu.VMEM((2, KG, CH), jnp.int32),
            pltpu.VMEM((2, KG, CH), jnp.int32),
            pltpu.VMEM_SHARED((NN + 8, PW), _f32),
        ] + [pltpu.SemaphoreType.DMA] * 8,
    )
    def k(hr, srcr, dstr, outr, bufs, sidx, didx, acc, *sems):
        c = lax.axis_index("c")
        s = lax.axis_index("s")
        srow = srcr.at[s]
        drow = dstr.at[s]
        r0 = s * RPT

        def one_pass(p, carry):
            q = 2 * p + c
            hqr = hr.at[q]
            oqr = outr.at[q]
            pltpu.sync_copy(hqr.at[pl.ds(r0, RPT)], acc.at[pl.ds(r0, RPT)])
            plsc.subcore_barrier()
            _edge_pipeline(hqr, srow, drow, acc, bufs, sidx, didx, sems, CPT)
            plsc.subcore_barrier()
            pltpu.sync_copy(acc.at[pl.ds(r0, RPT)], oqr.at[pl.ds(r0, RPT)])
            plsc.subcore_barrier()
            return carry

        lax.fori_loop(0, NPART // 2, one_pass, 0)

    return k(h_all, src_h, dst_h)


def _sc_agg_l0(x, z2, src_h, dst_h):
    @functools.partial(
        pl.kernel,
        out_type=[jax.ShapeDtypeStruct((NN, 16), _f32)] * 2,
        mesh=_sc_mesh(),
        compiler_params=pltpu.CompilerParams(use_tc_tiling_on_sc=False),
        scratch_types=[
            pltpu.VMEM((2, KG, CH, 16), _f32),
            pltpu.VMEM((2, KG, CH), jnp.int32),
            pltpu.VMEM((2, KG, CH), jnp.int32),
            pltpu.VMEM_SHARED((NN + 8, 16), _f32),
        ] + [pltpu.SemaphoreType.DMA] * 8,
    )
    def k(xr, zr, srcr, dstr, o0, o1, bufs, sidx, didx, acc, *sems):
        c = lax.axis_index("c")
        s = lax.axis_index("s")
        w = c * NS + s
        srow = srcr.at[w]
        drow = dstr.at[w]
        r0 = s * RPT
        for cc in range(2):
            init = (xr, zr)[cc]
            out = (o0, o1)[cc]

            @pl.when(c == cc)
            def _(init=init, out=out):
                pltpu.sync_copy(init.at[pl.ds(r0, RPT)], acc.at[pl.ds(r0, RPT)])
                plsc.subcore_barrier()
                _edge_pipeline(xr, srow, drow, acc, bufs, sidx, didx, sems, CPT0)
                plsc.subcore_barrier()
                pltpu.sync_copy(acc.at[pl.ds(r0, RPT)], out.at[pl.ds(r0, RPT)])

    return k(x, z2, src_h, dst_h)


def _full(shape):
    return pl.BlockSpec(shape, lambda i: tuple(0 for _ in shape))


def _sparts_specs(sparts):
    if isinstance(sparts, (list, tuple)):
        n = len(sparts)
        fin = sparts[0].shape[1]
        specs = [pl.BlockSpec((BN, fin), lambda i: (i, 0))] * n

        def read(refs):
            s = refs[0][...]
            for r in refs[1:]:
                s = s + r[...]
            return s

        return list(sparts), specs, n, read
    specs = [pl.BlockSpec((NPART, BN, PW), lambda i: (0, i, 0))]

    def read(refs):
        return jnp.concatenate([refs[0][qi] for qi in range(NPART)], axis=1)

    return [sparts], specs, 1, read


def _tc_stats(sparts, w1, b1):
    arrs, specs, n, read = _sparts_specs(sparts)

    def body(*refs):
        s = read(refs[:n])
        h1 = jnp.dot(s, refs[n][...], preferred_element_type=_f32) + refs[n + 1][...]
        ps = jnp.sum(h1, axis=0, keepdims=True)
        pq = jnp.sum(h1 * h1, axis=0, keepdims=True)
        stats = refs[n + 2]

        @pl.when(pl.program_id(0) == 0)
        def _():
            stats[...] = jnp.zeros_like(stats)

        stats[...] += jnp.concatenate([ps, pq, jnp.zeros((6, HH), _f32)], axis=0)

    return pl.pallas_call(
        body,
        grid=(NBLK,),
        in_specs=specs + [_full(w1.shape), _full((1, HH))],
        out_specs=_full((8, HH)),
        out_shape=jax.ShapeDtypeStruct((8, HH), _f32),
        compiler_params=pltpu.CompilerParams(dimension_semantics=("arbitrary",)),
    )(*arrs, w1, b1.reshape(1, HH))


def _bn_coeffs(statsr, gr, ber):
    st = statsr[...]
    mu = st[0:1, :] * (1.0 / NN)
    ex2 = st[1:2, :] * (1.0 / NN)
    var = ex2 - mu * mu
    sc = gr[...] * lax.rsqrt(var + 1e-5)
    sh = ber[...] - mu * sc
    return sc, sh


def _tc_mlp(sparts, stats, w1, b1, g, be, w2, b2):
    arrs, specs, n, read = _sparts_specs(sparts)

    def body(*refs):
        s = read(refs[:n])
        statsr, w1r, b1r, gr, ber, w2r, b2r, outr = refs[n:]
        sc, sh = _bn_coeffs(statsr, gr, ber)
        h1 = jnp.dot(s, w1r[...], preferred_element_type=_f32) + b1r[...]
        h = jnp.maximum(h1 * sc + sh, 0.0)
        h2 = jnp.maximum(jnp.dot(h, w2r[...], preferred_element_type=_f32) + b2r[...], 0.0)
        for qi in range(NPART):
            outr[qi] = h2[:, qi * PW:(qi + 1) * PW]

    return pl.pallas_call(
        body,
        grid=(NBLK,),
        in_specs=specs + [_full((8, HH)), _full(w1.shape), _full((1, HH)),
                          _full((1, HH)), _full((1, HH)), _full((HH, HH)),
                          _full((1, HH))],
        out_specs=pl.BlockSpec((NPART, BN, PW), lambda i: (0, i, 0)),
        out_shape=jax.ShapeDtypeStruct((NPART, NN, PW), _f32),
        compiler_params=pltpu.CompilerParams(dimension_semantics=("arbitrary",)),
    )(*arrs, stats, w1, b1.reshape(1, HH), g.reshape(1, HH), be.reshape(1, HH),
      w2, b2.reshape(1, HH))


def _tc_mlp_pool(sparts, stats, w1, b1, g, be, w2, b2, batch3, cls_w, cls_b):
    arrs, specs, n, read = _sparts_specs(sparts)

    def body(*refs):
        s = read(refs[:n])
        statsr, w1r, b1r, gr, ber, w2r, b2r, batchr, cwr, cbr, outv, pool = refs[n:]
        i = pl.program_id(0)
        sc, sh = _bn_coeffs(statsr, gr, ber)
        h1 = jnp.dot(s, w1r[...], preferred_element_type=_f32) + b1r[...]
        h = jnp.maximum(h1 * sc + sh, 0.0)
        h2 = jnp.maximum(jnp.dot(h, w2r[...], preferred_element_type=_f32) + b2r[...], 0.0)
        b = batchr[0, 0, :]
        onehot = (b[:, None] == lax.broadcasted_iota(jnp.int32, (BN, GG), 1)).astype(_f32)
        pblk = lax.dot_general(onehot, h2, (((0,), (0,)), ((), ())),
                               preferred_element_type=_f32)

        @pl.when(i == 0)
        def _():
            pool[...] = jnp.zeros_like(pool)

        pool[...] += pblk
        outv[...] = jnp.dot(pool[...], cwr[...], preferred_element_type=_f32) + cbr[...]

    return pl.pallas_call(
        body,
        grid=(NBLK,),
        in_specs=specs + [_full((8, HH)), _full((HH, HH)), _full((1, HH)),
                          _full((1, HH)), _full((1, HH)), _full((HH, HH)),
                          _full((1, HH)),
                          pl.BlockSpec((1, 1, BN), lambda i: (i, 0, 0)),
                          _full((HH, 1)), _full((1, 1))],
        out_specs=_full((GG, 1)),
        out_shape=jax.ShapeDtypeStruct((GG, 1), _f32),
        scratch_shapes=[pltpu.VMEM((GG, HH), _f32)],
        compiler_params=pltpu.CompilerParams(dimension_semantics=("arbitrary",)),
    )(*arrs, stats, w1, b1.reshape(1, HH), g.reshape(1, HH), be.reshape(1, HH),
      w2, b2.reshape(1, HH), batch3, cls_w, cls_b.reshape(1, 1))


def kernel(x, edge_index, batch, l0_w1, l0_b1, l0_g, l0_be, l0_w2, l0_b2,
           l1_w1, l1_b1, l1_g, l1_be, l1_w2, l1_b2,
           l2_w1, l2_b1, l2_g, l2_be, l2_w2, l2_b2, cls_w, cls_b):
    src = edge_index[0]
    dst = edge_index[1]
    i32 = jnp.int32
    src1 = jnp.concatenate([src, jnp.zeros((E_PAD1 - EE,), i32)]).reshape(NS, CPT, CH)
    dst1 = jnp.concatenate([dst, jnp.full((E_PAD1 - EE,), NN, i32)]).reshape(NS, CPT, CH)
    src0 = jnp.concatenate([src, jnp.zeros((E_PAD0 - EE,), i32)]).reshape(2 * NS, CPT0, CH)
    dst0 = jnp.concatenate([dst, jnp.full((E_PAD0 - EE,), NN, i32)]).reshape(2 * NS, CPT0, CH)
    x16 = jnp.pad(x, ((0, 0), (0, 14)))
    z16 = jnp.zeros((NN, 16), _f32)
    w1p = jnp.pad(l0_w1, ((0, 14), (0, 0)))
    batch3 = batch.reshape(NBLK, 1, BN)

    part0, part1 = _sc_agg_l0(x16, z16, src0, dst0)
    st0 = _tc_stats([part0, part1], w1p, l0_b1)
    h_all = _tc_mlp([part0, part1], st0, w1p, l0_b1, l0_g, l0_be, l0_w2, l0_b2)

    s_all = _sc_agg_parts(h_all, src1, dst1)
    st1 = _tc_stats(s_all, l1_w1, l1_b1)
    h_all = _tc_mlp(s_all, st1, l1_w1, l1_b1, l1_g, l1_be, l1_w2, l1_b2)

    s_all = _sc_agg_parts(h_all, src1, dst1)
    st2 = _tc_stats(s_all, l2_w1, l2_b1)
    outv = _tc_mlp_pool(s_all, st2, l2_w1, l2_b1, l2_g, l2_be, l2_w2, l2_b2,
                        batch3, cls_w, cls_b)
    return outv.reshape(-1)


# speedup vs baseline: 3.0120x; 1.0429x over previous
import functools

import jax
import jax.numpy as jnp
from jax import lax
from jax.experimental import pallas as pl
from jax.experimental.pallas import tpu as pltpu
from jax.experimental.pallas import tpu_sc as plsc

NN = 50000
EE = 800000
HH = 128
GG = 512

NC, NS = 2, 16
CH = 128
KG = 10
CPT = 400
E_PAD1 = NS * CPT * CH
CPT0 = 220
E_PAD0 = 2 * NS * CPT0 * CH
NPART = 8
PW = HH // NPART
RPT = NN // NS
NBLK = 50
BN = NN // NBLK

_f32 = jnp.float32


def _sc_mesh():
    return plsc.VectorSubcoreMesh(
        core_axis_name="c", subcore_axis_name="s", num_cores=NC, num_subcores=NS)


def _edge_pipeline(hqr, srow, drow, acc, bufs, sidx, didx, sems, n_chunks):
    gs0, gs1, ss0, ss1 = sems[:4]
    npairs = n_chunks // (2 * KG)

    def grp(gp, carry):
        ga = gp * 2
        isem = sems[4]
        dix = [pltpu.async_copy(srow.at[pl.ds(ga * KG, KG)], sidx.at[0], isem),
               pltpu.async_copy(srow.at[pl.ds((ga + 1) * KG, KG)], sidx.at[1], isem),
               pltpu.async_copy(drow.at[pl.ds(ga * KG, KG)], didx.at[0], isem),
               pltpu.async_copy(drow.at[pl.ds((ga + 1) * KG, KG)], didx.at[1], isem)]
        for d in dix:
            d.wait()
        dga = [pltpu.async_copy(hqr.at[sidx.at[0, b]], bufs.at[0, b], gs0)
               for b in range(KG)]
        dgb = [pltpu.async_copy(hqr.at[sidx.at[1, b]], bufs.at[1, b], gs1)
               for b in range(KG)]
        for d in dga:
            d.wait()
        dsa = [pltpu.async_copy(bufs.at[0, b], acc.at[didx.at[0, b]], ss0,
                                add=True) for b in range(KG)]
        for d in dgb:
            d.wait()
        dsb = [pltpu.async_copy(bufs.at[1, b], acc.at[didx.at[1, b]], ss1,
                                add=True) for b in range(KG)]
        for d in dsa:
            d.wait()
        for d in dsb:
            d.wait()
        return carry

    lax.fori_loop(0, npairs, grp, 0)


def _sc_agg_parts(h_all, src_h, dst_h):
    @functools.partial(
        pl.kernel,
        out_type=jax.ShapeDtypeStruct((NPART, NN, PW), _f32),
        mesh=_sc_mesh(),
        compiler_params=pltpu.CompilerParams(use_tc_tiling_on_sc=False),
        scratch_types=[
            pltpu.VMEM((2, KG, CH, PW), _f32),
            pltpu.VMEM((2, KG, CH), jnp.int32),
            pltpu.VMEM((2, KG, CH), jnp.int32),
            pltpu.VMEM_SHARED((NN + 8, PW), _f32),
        ] + [pltpu.SemaphoreType.DMA] * 8,
    )
    def k(hr, srcr, dstr, outr, bufs, sidx, didx, acc, *sems):
        c = lax.axis_index("c")
        s = lax.axis_index("s")
        srow = srcr.at[s]
        drow = dstr.at[s]
        r0 = s * RPT

        def one_pass(p, carry):
            q = 2 * p + c
            hqr = hr.at[q]
            oqr = outr.at[q]
            pltpu.sync_copy(hqr.at[pl.ds(r0, RPT)], acc.at[pl.ds(r0, RPT)])
            plsc.subcore_barrier()
            _edge_pipeline(hqr, srow, drow, acc, bufs, sidx, didx, sems, CPT)
            plsc.subcore_barrier()
            pltpu.sync_copy(acc.at[pl.ds(r0, RPT)], oqr.at[pl.ds(r0, RPT)])
            plsc.subcore_barrier()
            return carry

        lax.fori_loop(0, NPART // 2, one_pass, 0)

    return k(h_all, src_h, dst_h)


def _sc_agg_l0(x, z2, src_h, dst_h):
    @functools.partial(
        pl.kernel,
        out_type=[jax.ShapeDtypeStruct((NN, 16), _f32)] * 2,
        mesh=_sc_mesh(),
        compiler_params=pltpu.CompilerParams(use_tc_tiling_on_sc=False),
        scratch_types=[
            pltpu.VMEM((2, KG, CH, 16), _f32),
            pltpu.VMEM((2, KG, CH), jnp.int32),
            pltpu.VMEM((2, KG, CH), jnp.int32),
            pltpu.VMEM_SHARED((NN + 8, 16), _f32),
        ] + [pltpu.SemaphoreType.DMA] * 8,
    )
    def k(xr, zr, srcr, dstr, o0, o1, bufs, sidx, didx, acc, *sems):
        c = lax.axis_index("c")
        s = lax.axis_index("s")
        w = c * NS + s
        srow = srcr.at[w]
        drow = dstr.at[w]
        r0 = s * RPT
        for cc in range(2):
            init = (xr, zr)[cc]
            out = (o0, o1)[cc]

            @pl.when(c == cc)
            def _(init=init, out=out):
                pltpu.sync_copy(init.at[pl.ds(r0, RPT)], acc.at[pl.ds(r0, RPT)])
                plsc.subcore_barrier()
                _edge_pipeline(xr, srow, drow, acc, bufs, sidx, didx, sems, CPT0)
                plsc.subcore_barrier()
                pltpu.sync_copy(acc.at[pl.ds(r0, RPT)], out.at[pl.ds(r0, RPT)])

    return k(x, z2, src_h, dst_h)


def _full(shape):
    return pl.BlockSpec(shape, lambda i: tuple(0 for _ in shape))


def _sparts_specs(sparts):
    if isinstance(sparts, (list, tuple)):
        n = len(sparts)
        fin = sparts[0].shape[1]
        specs = [pl.BlockSpec((BN, fin), lambda i: (i, 0))] * n

        def read(refs):
            s = refs[0][...]
            for r in refs[1:]:
                s = s + r[...]
            return s

        return list(sparts), specs, n, read
    specs = [pl.BlockSpec((NPART, BN, PW), lambda i: (0, i, 0))]

    def read(refs):
        return jnp.concatenate([refs[0][qi] for qi in range(NPART)], axis=1)

    return [sparts], specs, 1, read


def _tc_stats(sparts, w1, b1):
    arrs, specs, n, read = _sparts_specs(sparts)

    def body(*refs):
        s = read(refs[:n])
        h1 = jnp.dot(s, refs[n][...], preferred_element_type=_f32) + refs[n + 1][...]
        ps = jnp.sum(h1, axis=0, keepdims=True)
        pq = jnp.sum(h1 * h1, axis=0, keepdims=True)
        stats = refs[n + 2]

        @pl.when(pl.program_id(0) == 0)
        def _():
            stats[...] = jnp.zeros_like(stats)

        stats[...] += jnp.concatenate([ps, pq, jnp.zeros((6, HH), _f32)], axis=0)

    return pl.pallas_call(
        body,
        grid=(NBLK,),
        in_specs=specs + [_full(w1.shape), _full((1, HH))],
        out_specs=_full((8, HH)),
        out_shape=jax.ShapeDtypeStruct((8, HH), _f32),
        compiler_params=pltpu.CompilerParams(dimension_semantics=("arbitrary",)),
    )(*arrs, w1, b1.reshape(1, HH))


def _bn_coeffs(statsr, gr, ber):
    st = statsr[...]
    mu = st[0:1, :] * (1.0 / NN)
    ex2 = st[1:2, :] * (1.0 / NN)
    var = ex2 - mu * mu
    sc = gr[...] * lax.rsqrt(var + 1e-5)
    sh = ber[...] - mu * sc
    return sc, sh


def _tc_mlp(sparts, stats, w1, b1, g, be, w2, b2):
    arrs, specs, n, read = _sparts_specs(sparts)

    def body(*refs):
        s = read(refs[:n])
        statsr, w1r, b1r, gr, ber, w2r, b2r, outr = refs[n:]
        sc, sh = _bn_coeffs(statsr, gr, ber)
        h1 = jnp.dot(s, w1r[...], preferred_element_type=_f32) + b1r[...]
        h = jnp.maximum(h1 * sc + sh, 0.0)
        h2 = jnp.maximum(jnp.dot(h, w2r[...], preferred_element_type=_f32) + b2r[...], 0.0)
        for qi in range(NPART):
            outr[qi] = h2[:, qi * PW:(qi + 1) * PW]

    return pl.pallas_call(
        body,
        grid=(NBLK,),
        in_specs=specs + [_full((8, HH)), _full(w1.shape), _full((1, HH)),
                          _full((1, HH)), _full((1, HH)), _full((HH, HH)),
                          _full((1, HH))],
        out_specs=pl.BlockSpec((NPART, BN, PW), lambda i: (0, i, 0)),
        out_shape=jax.ShapeDtypeStruct((NPART, NN, PW), _f32),
        compiler_params=pltpu.CompilerParams(dimension_semantics=("arbitrary",)),
    )(*arrs, stats, w1, b1.reshape(1, HH), g.reshape(1, HH), be.reshape(1, HH),
      w2, b2.reshape(1, HH))


def _tc_mlp_pool(sparts, stats, w1, b1, g, be, w2, b2, batch3, cls_w, cls_b):
    arrs, specs, n, read = _sparts_specs(sparts)

    def body(*refs):
        s = read(refs[:n])
        statsr, w1r, b1r, gr, ber, w2r, b2r, batchr, cwr, cbr, outv, pool = refs[n:]
        i = pl.program_id(0)
        sc, sh = _bn_coeffs(statsr, gr, ber)
        h1 = jnp.dot(s, w1r[...], preferred_element_type=_f32) + b1r[...]
        h = jnp.maximum(h1 * sc + sh, 0.0)
        h2 = jnp.maximum(jnp.dot(h, w2r[...], preferred_element_type=_f32) + b2r[...], 0.0)
        b = batchr[0, 0, :]
        onehot = (b[:, None] == lax.broadcasted_iota(jnp.int32, (BN, GG), 1)).astype(_f32)
        pblk = lax.dot_general(onehot, h2, (((0,), (0,)), ((), ())),
                               preferred_element_type=_f32)

        @pl.when(i == 0)
        def _():
            pool[...] = jnp.zeros_like(pool)

        pool[...] += pblk
        outv[...] = jnp.dot(pool[...], cwr[...], preferred_element_type=_f32) + cbr[...]

    return pl.pallas_call(
        body,
        grid=(NBLK,),
        in_specs=specs + [_full((8, HH)), _full((HH, HH)), _full((1, HH)),
                          _full((1, HH)), _full((1, HH)), _full((HH, HH)),
                          _full((1, HH)),
                          pl.BlockSpec((1, 1, BN), lambda i: (i, 0, 0)),
                          _full((HH, 1)), _full((1, 1))],
        out_specs=_full((GG, 1)),
        out_shape=jax.ShapeDtypeStruct((GG, 1), _f32),
        scratch_shapes=[pltpu.VMEM((GG, HH), _f32)],
        compiler_params=pltpu.CompilerParams(dimension_semantics=("arbitrary",)),
    )(*arrs, stats, w1, b1.reshape(1, HH), g.reshape(1, HH), be.reshape(1, HH),
      w2, b2.reshape(1, HH), batch3, cls_w, cls_b.reshape(1, 1))


def kernel(x, edge_index, batch, l0_w1, l0_b1, l0_g, l0_be, l0_w2, l0_b2,
           l1_w1, l1_b1, l1_g, l1_be, l1_w2, l1_b2,
           l2_w1, l2_b1, l2_g, l2_be, l2_w2, l2_b2, cls_w, cls_b):
    src = edge_index[0]
    dst = edge_index[1]
    i32 = jnp.int32
    src1 = jnp.concatenate([src, jnp.zeros((E_PAD1 - EE,), i32)]).reshape(NS, CPT, CH)
    dst1 = jnp.concatenate([dst, jnp.full((E_PAD1 - EE,), NN, i32)]).reshape(NS, CPT, CH)
    src0 = jnp.concatenate([src, jnp.zeros((E_PAD0 - EE,), i32)]).reshape(2 * NS, CPT0, CH)
    dst0 = jnp.concatenate([dst, jnp.full((E_PAD0 - EE,), NN, i32)]).reshape(2 * NS, CPT0, CH)
    x16 = jnp.pad(x, ((0, 0), (0, 14)))
    z16 = jnp.zeros((NN, 16), _f32)
    w1p = jnp.pad(l0_w1, ((0, 14), (0, 0)))
    batch3 = batch.reshape(NBLK, 1, BN)

    part0, part1 = _sc_agg_l0(x16, z16, src0, dst0)
    st0 = _tc_stats([part0, part1], w1p, l0_b1)
    h_all = _tc_mlp([part0, part1], st0, w1p, l0_b1, l0_g, l0_be, l0_w2, l0_b2)

    s_all = _sc_agg_parts(h_all, src1, dst1)
    st1 = _tc_stats(s_all, l1_w1, l1_b1)
    h_all = _tc_mlp(s_all, st1, l1_w1, l1_b1, l1_g, l1_be, l1_w2, l1_b2)

    s_all = _sc_agg_parts(h_all, src1, dst1)
    st2 = _tc_stats(s_all, l2_w1, l2_b1)
    outv = _tc_mlp_pool(s_all, st2, l2_w1, l2_b1, l2_g, l2_be, l2_w2, l2_b2,
                        batch3, cls_w, cls_b)
    return outv.reshape(-1)


# speedup vs baseline: 3.2958x; 1.0942x over previous
import functools

import jax
import jax.numpy as jnp
from jax import lax
from jax.experimental import pallas as pl
from jax.experimental.pallas import tpu as pltpu
from jax.experimental.pallas import tpu_sc as plsc

NN = 50000
EE = 800000
HH = 128
GG = 512

NC, NS = 2, 16
CH = 128
KG = 3
CPT = 402
E_PAD1 = NS * CPT * CH
CPT0 = 216
E_PAD0 = 2 * NS * CPT0 * CH
NPART = 4
PW = HH // NPART
RPT = NN // NS
NBLK = 50
BN = NN // NBLK

_f32 = jnp.float32


def _sc_mesh():
    return plsc.VectorSubcoreMesh(
        core_axis_name="c", subcore_axis_name="s", num_cores=NC, num_subcores=NS)


def _edge_pipeline(hqr, srow, drow, acc, bufs, sidx, didx, sems, n_chunks):
    gs0, gs1, ss0, ss1 = sems[:4]
    npairs = n_chunks // (2 * KG)

    def grp(gp, carry):
        ga = gp * 2
        isem = sems[4]
        dix = [pltpu.async_copy(srow.at[pl.ds(ga * KG, KG)], sidx.at[0], isem),
               pltpu.async_copy(srow.at[pl.ds((ga + 1) * KG, KG)], sidx.at[1], isem),
               pltpu.async_copy(drow.at[pl.ds(ga * KG, KG)], didx.at[0], isem),
               pltpu.async_copy(drow.at[pl.ds((ga + 1) * KG, KG)], didx.at[1], isem)]
        for d in dix:
            d.wait()
        dga = [pltpu.async_copy(hqr.at[sidx.at[0, b]], bufs.at[0, b], gs0)
               for b in range(KG)]
        dgb = [pltpu.async_copy(hqr.at[sidx.at[1, b]], bufs.at[1, b], gs1)
               for b in range(KG)]
        for d in dga:
            d.wait()
        dsa = [pltpu.async_copy(bufs.at[0, b], acc.at[didx.at[0, b]], ss0,
                                add=True) for b in range(KG)]
        for d in dgb:
            d.wait()
        dsb = [pltpu.async_copy(bufs.at[1, b], acc.at[didx.at[1, b]], ss1,
                                add=True) for b in range(KG)]
        for d in dsa:
            d.wait()
        for d in dsb:
            d.wait()
        return carry

    lax.fori_loop(0, npairs, grp, 0)


def _sc_agg_parts(h_all, src_h, dst_h):
    @functools.partial(
        pl.kernel,
        out_type=jax.ShapeDtypeStruct((NPART, NN, PW), _f32),
        mesh=_sc_mesh(),
        compiler_params=pltpu.CompilerParams(use_tc_tiling_on_sc=False),
        scratch_types=[
            pltpu.VMEM((2, KG, CH, PW), _f32),
            pltpu.VMEM((2, KG, CH), jnp.int32),
            pltpu.VMEM((2, KG, CH), jnp.int32),
            pltpu.VMEM_SHARED((NN + 8, PW), _f32),
        ] + [pltpu.SemaphoreType.DMA] * 8,
    )
    def k(hr, srcr, dstr, outr, bufs, sidx, didx, acc, *sems):
        c = lax.axis_index("c")
        s = lax.axis_index("s")
        srow = srcr.at[s]
        drow = dstr.at[s]
        r0 = s * RPT

        def one_pass(p, carry):
            q = 2 * p + c
            hqr = hr.at[q]
            oqr = outr.at[q]
            pltpu.sync_copy(hqr.at[pl.ds(r0, RPT)], acc.at[pl.ds(r0, RPT)])
            plsc.subcore_barrier()
            _edge_pipeline(hqr, srow, drow, acc, bufs, sidx, didx, sems, CPT)
            plsc.subcore_barrier()
            pltpu.sync_copy(acc.at[pl.ds(r0, RPT)], oqr.at[pl.ds(r0, RPT)])
            plsc.subcore_barrier()
            return carry

        lax.fori_loop(0, NPART // 2, one_pass, 0)

    return k(h_all, src_h, dst_h)


def _sc_agg_l0(x, z2, src_h, dst_h):
    @functools.partial(
        pl.kernel,
        out_type=[jax.ShapeDtypeStruct((NN, 16), _f32)] * 2,
        mesh=_sc_mesh(),
        compiler_params=pltpu.CompilerParams(use_tc_tiling_on_sc=False),
        scratch_types=[
            pltpu.VMEM((2, KG, CH, 16), _f32),
            pltpu.VMEM((2, KG, CH), jnp.int32),
            pltpu.VMEM((2, KG, CH), jnp.int32),
            pltpu.VMEM_SHARED((NN + 8, 16), _f32),
        ] + [pltpu.SemaphoreType.DMA] * 8,
    )
    def k(xr, zr, srcr, dstr, o0, o1, bufs, sidx, didx, acc, *sems):
        c = lax.axis_index("c")
        s = lax.axis_index("s")
        w = c * NS + s
        srow = srcr.at[w]
        drow = dstr.at[w]
        r0 = s * RPT
        for cc in range(2):
            init = (xr, zr)[cc]
            out = (o0, o1)[cc]

            @pl.when(c == cc)
            def _(init=init, out=out):
                pltpu.sync_copy(init.at[pl.ds(r0, RPT)], acc.at[pl.ds(r0, RPT)])
                plsc.subcore_barrier()
                _edge_pipeline(xr, srow, drow, acc, bufs, sidx, didx, sems, CPT0)
                plsc.subcore_barrier()
                pltpu.sync_copy(acc.at[pl.ds(r0, RPT)], out.at[pl.ds(r0, RPT)])

    return k(x, z2, src_h, dst_h)


def _full(shape):
    return pl.BlockSpec(shape, lambda i: tuple(0 for _ in shape))


def _sparts_specs(sparts):
    if isinstance(sparts, (list, tuple)):
        n = len(sparts)
        fin = sparts[0].shape[1]
        specs = [pl.BlockSpec((BN, fin), lambda i: (i, 0))] * n

        def read(refs):
            s = refs[0][...]
            for r in refs[1:]:
                s = s + r[...]
            return s

        return list(sparts), specs, n, read
    specs = [pl.BlockSpec((NPART, BN, PW), lambda i: (0, i, 0))]

    def read(refs):
        return jnp.concatenate([refs[0][qi] for qi in range(NPART)], axis=1)

    return [sparts], specs, 1, read


def _tc_stats(sparts, w1, b1):
    arrs, specs, n, read = _sparts_specs(sparts)

    def body(*refs):
        s = read(refs[:n])
        h1 = jnp.dot(s, refs[n][...], preferred_element_type=_f32) + refs[n + 1][...]
        ps = jnp.sum(h1, axis=0, keepdims=True)
        pq = jnp.sum(h1 * h1, axis=0, keepdims=True)
        stats = refs[n + 2]

        @pl.when(pl.program_id(0) == 0)
        def _():
            stats[...] = jnp.zeros_like(stats)

        stats[...] += jnp.concatenate([ps, pq, jnp.zeros((6, HH), _f32)], axis=0)

    return pl.pallas_call(
        body,
        grid=(NBLK,),
        in_specs=specs + [_full(w1.shape), _full((1, HH))],
        out_specs=_full((8, HH)),
        out_shape=jax.ShapeDtypeStruct((8, HH), _f32),
        compiler_params=pltpu.CompilerParams(dimension_semantics=("arbitrary",)),
    )(*arrs, w1, b1.reshape(1, HH))


def _bn_coeffs(statsr, gr, ber):
    st = statsr[...]
    mu = st[0:1, :] * (1.0 / NN)
    ex2 = st[1:2, :] * (1.0 / NN)
    var = ex2 - mu * mu
    sc = gr[...] * lax.rsqrt(var + 1e-5)
    sh = ber[...] - mu * sc
    return sc, sh


def _tc_mlp(sparts, stats, w1, b1, g, be, w2, b2):
    arrs, specs, n, read = _sparts_specs(sparts)

    def body(*refs):
        s = read(refs[:n])
        statsr, w1r, b1r, gr, ber, w2r, b2r, outr = refs[n:]
        sc, sh = _bn_coeffs(statsr, gr, ber)
        h1 = jnp.dot(s, w1r[...], preferred_element_type=_f32) + b1r[...]
        h = jnp.maximum(h1 * sc + sh, 0.0)
        h2 = jnp.maximum(jnp.dot(h, w2r[...], preferred_element_type=_f32) + b2r[...], 0.0)
        for qi in range(NPART):
            outr[qi] = h2[:, qi * PW:(qi + 1) * PW]

    return pl.pallas_call(
        body,
        grid=(NBLK,),
        in_specs=specs + [_full((8, HH)), _full(w1.shape), _full((1, HH)),
                          _full((1, HH)), _full((1, HH)), _full((HH, HH)),
                          _full((1, HH))],
        out_specs=pl.BlockSpec((NPART, BN, PW), lambda i: (0, i, 0)),
        out_shape=jax.ShapeDtypeStruct((NPART, NN, PW), _f32),
        compiler_params=pltpu.CompilerParams(dimension_semantics=("arbitrary",)),
    )(*arrs, stats, w1, b1.reshape(1, HH), g.reshape(1, HH), be.reshape(1, HH),
      w2, b2.reshape(1, HH))


def _tc_mlp_pool(sparts, stats, w1, b1, g, be, w2, b2, batch3, cls_w, cls_b):
    arrs, specs, n, read = _sparts_specs(sparts)

    def body(*refs):
        s = read(refs[:n])
        statsr, w1r, b1r, gr, ber, w2r, b2r, batchr, cwr, cbr, outv, pool = refs[n:]
        i = pl.program_id(0)
        sc, sh = _bn_coeffs(statsr, gr, ber)
        h1 = jnp.dot(s, w1r[...], preferred_element_type=_f32) + b1r[...]
        h = jnp.maximum(h1 * sc + sh, 0.0)
        h2 = jnp.maximum(jnp.dot(h, w2r[...], preferred_element_type=_f32) + b2r[...], 0.0)
        b = batchr[0, 0, :]
        onehot = (b[:, None] == lax.broadcasted_iota(jnp.int32, (BN, GG), 1)).astype(_f32)
        pblk = lax.dot_general(onehot, h2, (((0,), (0,)), ((), ())),
                               preferred_element_type=_f32)

        @pl.when(i == 0)
        def _():
            pool[...] = jnp.zeros_like(pool)

        pool[...] += pblk
        outv[...] = jnp.dot(pool[...], cwr[...], preferred_element_type=_f32) + cbr[...]

    return pl.pallas_call(
        body,
        grid=(NBLK,),
        in_specs=specs + [_full((8, HH)), _full((HH, HH)), _full((1, HH)),
                          _full((1, HH)), _full((1, HH)), _full((HH, HH)),
                          _full((1, HH)),
                          pl.BlockSpec((1, 1, BN), lambda i: (i, 0, 0)),
                          _full((HH, 1)), _full((1, 1))],
        out_specs=_full((GG, 1)),
        out_shape=jax.ShapeDtypeStruct((GG, 1), _f32),
        scratch_shapes=[pltpu.VMEM((GG, HH), _f32)],
        compiler_params=pltpu.CompilerParams(dimension_semantics=("arbitrary",)),
    )(*arrs, stats, w1, b1.reshape(1, HH), g.reshape(1, HH), be.reshape(1, HH),
      w2, b2.reshape(1, HH), batch3, cls_w, cls_b.reshape(1, 1))


def kernel(x, edge_index, batch, l0_w1, l0_b1, l0_g, l0_be, l0_w2, l0_b2,
           l1_w1, l1_b1, l1_g, l1_be, l1_w2, l1_b2,
           l2_w1, l2_b1, l2_g, l2_be, l2_w2, l2_b2, cls_w, cls_b):
    src = edge_index[0]
    dst = edge_index[1]
    i32 = jnp.int32
    src1 = jnp.concatenate([src, jnp.zeros((E_PAD1 - EE,), i32)]).reshape(NS, CPT, CH)
    dst1 = jnp.concatenate([dst, jnp.full((E_PAD1 - EE,), NN, i32)]).reshape(NS, CPT, CH)
    src0 = jnp.concatenate([src, jnp.zeros((E_PAD0 - EE,), i32)]).reshape(2 * NS, CPT0, CH)
    dst0 = jnp.concatenate([dst, jnp.full((E_PAD0 - EE,), NN, i32)]).reshape(2 * NS, CPT0, CH)
    x16 = jnp.pad(x, ((0, 0), (0, 14)))
    z16 = jnp.zeros((NN, 16), _f32)
    w1p = jnp.pad(l0_w1, ((0, 14), (0, 0)))
    batch3 = batch.reshape(NBLK, 1, BN)

    part0, part1 = _sc_agg_l0(x16, z16, src0, dst0)
    st0 = _tc_stats([part0, part1], w1p, l0_b1)
    h_all = _tc_mlp([part0, part1], st0, w1p, l0_b1, l0_g, l0_be, l0_w2, l0_b2)

    s_all = _sc_agg_parts(h_all, src1, dst1)
    st1 = _tc_stats(s_all, l1_w1, l1_b1)
    h_all = _tc_mlp(s_all, st1, l1_w1, l1_b1, l1_g, l1_be, l1_w2, l1_b2)

    s_all = _sc_agg_parts(h_all, src1, dst1)
    st2 = _tc_stats(s_all, l2_w1, l2_b1)
    outv = _tc_mlp_pool(s_all, st2, l2_w1, l2_b1, l2_g, l2_be, l2_w2, l2_b2,
                        batch3, cls_w, cls_b)
    return outv.reshape(-1)
